# baseline (device time: 20575 ns/iter reference)
import jax
import jax.numpy as jnp
from jax import lax
from jax.experimental import pallas as pl
from jax.experimental.pallas import tpu as pltpu

N_DEV = 4


def kernel(x):
    m, n = x.shape
    half = m // 2
    q = m // 4
    e = m // 8

    def body(x_ref, out_ref, xb, bufA1, bufB1, bufA2, bufB2, accA, accB,
             send_sems, recv_sems):
        p = lax.axis_index("i")
        gx = p // 2
        gy = gx ^ (p % 2)
        px = 3 - p
        py = p ^ 1

        def copy(src, dst, sem, dev):
            return pltpu.make_async_remote_copy(
                src_ref=src, dst_ref=dst,
                send_sem=send_sems.at[sem], recv_sem=recv_sems.at[sem],
                device_id=(dev,), device_id_type=pl.DeviceIdType.MESH,
            )

        barrier_sem = pltpu.get_barrier_semaphore()
        for nbr in (px, py):
            pl.semaphore_signal(
                barrier_sem, inc=1,
                device_id=(nbr,), device_id_type=pl.DeviceIdType.MESH,
            )
        xb[:, :] = x_ref[:, :].astype(jnp.bfloat16)
        pl.semaphore_wait(barrier_sem, 2)

        a1 = copy(xb.at[pl.ds((1 - gx) * q, q), :], bufA1, 0, px)
        b1 = copy(xb.at[pl.ds(half + (1 - gy) * q, q), :], bufB1, 1, py)
        a1.start()
        b1.start()

        a1.wait_recv()
        accA[:, :] = xb[pl.ds(gx * q, q), :] + bufA1[:, :]
        a2 = copy(accA.at[pl.ds((1 - gy) * e, e), :], bufA2, 2, py)
        a2.start()

        b1.wait_recv()
        accB[:, :] = xb[pl.ds(half + gy * q, q), :] + bufB1[:, :]
        b2 = copy(accB.at[pl.ds((1 - gx) * e, e), :], bufB2, 3, px)
        b2.start()

        a2.wait_recv()
        rowsA = pl.ds(gx * q + gy * e, e)
        out_ref[rowsA, :] = accA[pl.ds(gy * e, e), :] + bufA2[:, :]
        a3 = copy(out_ref.at[rowsA, :], out_ref.at[rowsA, :], 4, py)
        a3.start()

        b2.wait_recv()
        rowsB = pl.ds(half + gy * q + gx * e, e)
        out_ref[rowsB, :] = accB[pl.ds(gx * e, e), :] + bufB2[:, :]
        b3 = copy(out_ref.at[rowsB, :], out_ref.at[rowsB, :], 5, px)
        b3.start()

        a3.wait_recv()
        quarterA = pl.ds(gx * q, q)
        a4 = copy(out_ref.at[quarterA, :], out_ref.at[quarterA, :], 6, px)
        a4.start()

        b3.wait_recv()
        quarterB = pl.ds(half + gy * q, q)
        b4 = copy(out_ref.at[quarterB, :], out_ref.at[quarterB, :], 7, py)
        b4.start()

        a4.wait_recv()
        b4.wait_recv()

        for rdma in (a1, b1, a2, b2, a3, b3, a4, b4):
            rdma.wait_send()

    return pl.pallas_call(
        body,
        out_shape=jax.ShapeDtypeStruct((m, n), jnp.bfloat16),
        in_specs=[pl.BlockSpec(memory_space=pltpu.VMEM)],
        out_specs=pl.BlockSpec(memory_space=pltpu.VMEM),
        scratch_shapes=[
            pltpu.VMEM((m, n), jnp.bfloat16),
            pltpu.VMEM((q, n), jnp.bfloat16),
            pltpu.VMEM((q, n), jnp.bfloat16),
            pltpu.VMEM((e, n), jnp.bfloat16),
            pltpu.VMEM((e, n), jnp.bfloat16),
            pltpu.VMEM((q, n), jnp.bfloat16),
            pltpu.VMEM((q, n), jnp.bfloat16),
            pltpu.SemaphoreType.DMA((8,)),
            pltpu.SemaphoreType.DMA((8,)),
        ],
        compiler_params=pltpu.CompilerParams(collective_id=0),
    )(x)


# device time: 18859 ns/iter; 1.0910x vs baseline; 1.0910x over previous
import jax
import jax.numpy as jnp
from jax import lax
from jax.experimental import pallas as pl
from jax.experimental.pallas import tpu as pltpu

N_DEV = 4
S = 2


def kernel(x):
    m, n = x.shape
    ch = m // N_DEV
    sub = ch // S

    def body(x_ref, out_ref, xb, rs_buf, ag_src,
             rs_send_sems, rs_recv_sems, ag_send_sems, ag_recv_sems):
        my = lax.axis_index("i")

        barrier_sem = pltpu.get_barrier_semaphore()
        for k in range(1, N_DEV):
            pl.semaphore_signal(
                barrier_sem, inc=1,
                device_id=((my + k) % N_DEV,),
                device_id_type=pl.DeviceIdType.MESH,
            )
        xb[:, :] = x_ref[:, :].astype(jnp.bfloat16)
        pl.semaphore_wait(barrier_sem, N_DEV - 1)

        rs_rdmas = {}
        for s in range(S):
            for k in (2, 1, 3):
                dst = (my + k) % N_DEV
                rdma = pltpu.make_async_remote_copy(
                    src_ref=xb.at[pl.ds(dst * ch + s * sub, sub), :],
                    dst_ref=rs_buf.at[k - 1, pl.ds(s * sub, sub), :],
                    send_sem=rs_send_sems.at[k - 1, s],
                    recv_sem=rs_recv_sems.at[k - 1, s],
                    device_id=(dst,),
                    device_id_type=pl.DeviceIdType.MESH,
                )
                rdma.start()
                rs_rdmas[(k, s)] = rdma

        ag_rdmas = []
        for s in range(S):
            for k in range(1, N_DEV):
                rs_rdmas[(k, s)].wait_recv()
            rows = pl.ds(s * sub, sub)
            ag_src[rows, :] = (
                xb[pl.ds(my * ch + s * sub, sub), :]
                + rs_buf[0, rows, :]
                + rs_buf[1, rows, :]
                + rs_buf[2, rows, :]
            )
            for k in (2, 1, 3):
                dst = (my + k) % N_DEV
                rdma = pltpu.make_async_remote_copy(
                    src_ref=ag_src.at[rows, :],
                    dst_ref=out_ref.at[pl.ds(my * ch + s * sub, sub), :],
                    send_sem=ag_send_sems.at[k - 1, s],
                    recv_sem=ag_recv_sems.at[k - 1, s],
                    device_id=(dst,),
                    device_id_type=pl.DeviceIdType.MESH,
                )
                rdma.start()
                ag_rdmas.append(rdma)

        out_ref[pl.ds(my * ch, ch), :] = ag_src[:, :]

        for rdma in ag_rdmas:
            rdma.wait_recv()

        for rdma in rs_rdmas.values():
            rdma.wait_send()
        for rdma in ag_rdmas:
            rdma.wait_send()

    return pl.pallas_call(
        body,
        out_shape=jax.ShapeDtypeStruct((m, n), jnp.bfloat16),
        in_specs=[pl.BlockSpec(memory_space=pltpu.VMEM)],
        out_specs=pl.BlockSpec(memory_space=pltpu.VMEM),
        scratch_shapes=[
            pltpu.VMEM((m, n), jnp.bfloat16),
            pltpu.VMEM((N_DEV - 1, ch, n), jnp.bfloat16),
            pltpu.VMEM((ch, n), jnp.bfloat16),
            pltpu.SemaphoreType.DMA((N_DEV - 1, S)),
            pltpu.SemaphoreType.DMA((N_DEV - 1, S)),
            pltpu.SemaphoreType.DMA((N_DEV - 1, S)),
            pltpu.SemaphoreType.DMA((N_DEV - 1, S)),
        ],
        compiler_params=pltpu.CompilerParams(collective_id=0),
    )(x)


# device time: 18080 ns/iter; 1.1380x vs baseline; 1.0431x over previous
import jax
import jax.numpy as jnp
from jax import lax
from jax.experimental import pallas as pl
from jax.experimental.pallas import tpu as pltpu

N_DEV = 4
G = 2

A1, B1, A2, B2, A3, B3, A4, B4 = range(8)


def kernel(x):
    m, n = x.shape
    half = m // 2
    q = m // 4
    e = m // 8
    cw = n // G

    def body(x_ref, out_ref, xb, bufA1, bufB1, bufA2, bufB2, accA, accB,
             send_sems, recv_sems):
        p = lax.axis_index("i")
        gx = p // 2
        gy = gx ^ (p % 2)
        px = 3 - p
        py = p ^ 1

        cols = [pl.ds(g * cw, cw) for g in range(G)]
        rowsA = pl.ds(gx * q + gy * e, e)
        rowsB = pl.ds(half + gy * q + gx * e, e)
        quarterA = pl.ds(gx * q, q)
        quarterB = pl.ds(half + gy * q, q)

        def copy(src, dst, stage, g, dev):
            return pltpu.make_async_remote_copy(
                src_ref=src, dst_ref=dst,
                send_sem=send_sems.at[stage, g],
                recv_sem=recv_sems.at[stage, g],
                device_id=(dev,), device_id_type=pl.DeviceIdType.MESH,
            )

        barrier_sem = pltpu.get_barrier_semaphore()
        for nbr in (px, py):
            pl.semaphore_signal(
                barrier_sem, inc=1,
                device_id=(nbr,), device_id_type=pl.DeviceIdType.MESH,
            )
        xb[:, :] = x_ref[:, :].astype(jnp.bfloat16)
        pl.semaphore_wait(barrier_sem, 2)

        a1, b1, a2, b2, a3, b3, a4, b4 = ({} for _ in range(8))
        for g in range(G):
            a1[g] = copy(xb.at[pl.ds((1 - gx) * q, q), cols[g]],
                         bufA1.at[:, cols[g]], A1, g, px)
            b1[g] = copy(xb.at[pl.ds(half + (1 - gy) * q, q), cols[g]],
                         bufB1.at[:, cols[g]], B1, g, py)
            a1[g].start()
            b1[g].start()

        for g in range(G):
            a1[g].wait_recv()
            accA[:, cols[g]] = xb[pl.ds(gx * q, q), cols[g]] + bufA1[:, cols[g]]
            a2[g] = copy(accA.at[pl.ds((1 - gy) * e, e), cols[g]],
                         bufA2.at[:, cols[g]], A2, g, py)
            a2[g].start()

            b1[g].wait_recv()
            accB[:, cols[g]] = (
                xb[pl.ds(half + gy * q, q), cols[g]] + bufB1[:, cols[g]]
            )
            b2[g] = copy(accB.at[pl.ds((1 - gx) * e, e), cols[g]],
                         bufB2.at[:, cols[g]], B2, g, px)
            b2[g].start()

        for g in range(G):
            a2[g].wait_recv()
            out_ref[rowsA, cols[g]] = (
                accA[pl.ds(gy * e, e), cols[g]] + bufA2[:, cols[g]]
            )
            a3[g] = copy(out_ref.at[rowsA, cols[g]],
                         out_ref.at[rowsA, cols[g]], A3, g, py)
            a3[g].start()

            b2[g].wait_recv()
            out_ref[rowsB, cols[g]] = (
                accB[pl.ds(gx * e, e), cols[g]] + bufB2[:, cols[g]]
            )
            b3[g] = copy(out_ref.at[rowsB, cols[g]],
                         out_ref.at[rowsB, cols[g]], B3, g, px)
            b3[g].start()

        for g in range(G):
            a3[g].wait_recv()
            a4[g] = copy(out_ref.at[quarterA, cols[g]],
                         out_ref.at[quarterA, cols[g]], A4, g, px)
            a4[g].start()

            b3[g].wait_recv()
            b4[g] = copy(out_ref.at[quarterB, cols[g]],
                         out_ref.at[quarterB, cols[g]], B4, g, py)
            b4[g].start()

        for g in range(G):
            a4[g].wait_recv()
            b4[g].wait_recv()

        for d in (a1, b1, a2, b2, a3, b3, a4, b4):
            for g in range(G):
                d[g].wait_send()

    return pl.pallas_call(
        body,
        out_shape=jax.ShapeDtypeStruct((m, n), jnp.bfloat16),
        in_specs=[pl.BlockSpec(memory_space=pltpu.VMEM)],
        out_specs=pl.BlockSpec(memory_space=pltpu.VMEM),
        scratch_shapes=[
            pltpu.VMEM((m, n), jnp.bfloat16),
            pltpu.VMEM((q, n), jnp.bfloat16),
            pltpu.VMEM((q, n), jnp.bfloat16),
            pltpu.VMEM((e, n), jnp.bfloat16),
            pltpu.VMEM((e, n), jnp.bfloat16),
            pltpu.VMEM((q, n), jnp.bfloat16),
            pltpu.VMEM((q, n), jnp.bfloat16),
            pltpu.SemaphoreType.DMA((8, G)),
            pltpu.SemaphoreType.DMA((8, G)),
        ],
        compiler_params=pltpu.CompilerParams(collective_id=0),
    )(x)


# device time: 17687 ns/iter; 1.1633x vs baseline; 1.0222x over previous
import jax
import jax.numpy as jnp
from jax import lax
from jax.experimental import pallas as pl
from jax.experimental.pallas import tpu as pltpu

N_DEV = 4
G = 4

A1, B1, A2, B2, A3, B3, A4, B4 = range(8)


def kernel(x):
    m, n = x.shape
    half = m // 2
    q = m // 4
    e = m // 8
    cw = n // G

    def body(x_ref, out_ref, xb, bufA1, bufB1, bufA2, bufB2, accA, accB,
             send_sems, recv_sems):
        p = lax.axis_index("i")
        gx = p // 2
        gy = gx ^ (p % 2)
        px = 3 - p
        py = p ^ 1

        cols = [pl.ds(g * cw, cw) for g in range(G)]
        rowsA = pl.ds(gx * q + gy * e, e)
        rowsB = pl.ds(half + gy * q + gx * e, e)
        quarterA = pl.ds(gx * q, q)
        quarterB = pl.ds(half + gy * q, q)

        def copy(src, dst, stage, g, dev):
            return pltpu.make_async_remote_copy(
                src_ref=src, dst_ref=dst,
                send_sem=send_sems.at[stage, g],
                recv_sem=recv_sems.at[stage, g],
                device_id=(dev,), device_id_type=pl.DeviceIdType.MESH,
            )

        barrier_sem = pltpu.get_barrier_semaphore()
        for nbr in (px, py):
            pl.semaphore_signal(
                barrier_sem, inc=1,
                device_id=(nbr,), device_id_type=pl.DeviceIdType.MESH,
            )
        xb[:, :] = x_ref[:, :].astype(jnp.bfloat16)
        pl.semaphore_wait(barrier_sem, 2)

        a1, b1, a2, b2, a3, b3, a4, b4 = ({} for _ in range(8))
        for g in range(G):
            a1[g] = copy(xb.at[pl.ds((1 - gx) * q, q), cols[g]],
                         bufA1.at[:, cols[g]], A1, g, px)
            b1[g] = copy(xb.at[pl.ds(half + (1 - gy) * q, q), cols[g]],
                         bufB1.at[:, cols[g]], B1, g, py)
            a1[g].start()
            b1[g].start()

        for g in range(G):
            a1[g].wait_recv()
            accA[:, cols[g]] = xb[pl.ds(gx * q, q), cols[g]] + bufA1[:, cols[g]]
            a2[g] = copy(accA.at[pl.ds((1 - gy) * e, e), cols[g]],
                         bufA2.at[:, cols[g]], A2, g, py)
            a2[g].start()

            b1[g].wait_recv()
            accB[:, cols[g]] = (
                xb[pl.ds(half + gy * q, q), cols[g]] + bufB1[:, cols[g]]
            )
            b2[g] = copy(accB.at[pl.ds((1 - gx) * e, e), cols[g]],
                         bufB2.at[:, cols[g]], B2, g, px)
            b2[g].start()

        for g in range(G):
            a2[g].wait_recv()
            out_ref[rowsA, cols[g]] = (
                accA[pl.ds(gy * e, e), cols[g]] + bufA2[:, cols[g]]
            )
            a3[g] = copy(out_ref.at[rowsA, cols[g]],
                         out_ref.at[rowsA, cols[g]], A3, g, py)
            a3[g].start()

            b2[g].wait_recv()
            out_ref[rowsB, cols[g]] = (
                accB[pl.ds(gx * e, e), cols[g]] + bufB2[:, cols[g]]
            )
            b3[g] = copy(out_ref.at[rowsB, cols[g]],
                         out_ref.at[rowsB, cols[g]], B3, g, px)
            b3[g].start()

        for g in range(G):
            a3[g].wait_recv()
            a4[g] = copy(out_ref.at[quarterA, cols[g]],
                         out_ref.at[quarterA, cols[g]], A4, g, px)
            a4[g].start()

            b3[g].wait_recv()
            b4[g] = copy(out_ref.at[quarterB, cols[g]],
                         out_ref.at[quarterB, cols[g]], B4, g, py)
            b4[g].start()

        for g in range(G):
            a4[g].wait_recv()
            b4[g].wait_recv()

        for d in (a1, b1, a2, b2, a3, b3, a4, b4):
            for g in range(G):
                d[g].wait_send()

    return pl.pallas_call(
        body,
        out_shape=jax.ShapeDtypeStruct((m, n), jnp.bfloat16),
        in_specs=[pl.BlockSpec(memory_space=pltpu.VMEM)],
        out_specs=pl.BlockSpec(memory_space=pltpu.VMEM),
        scratch_shapes=[
            pltpu.VMEM((m, n), jnp.bfloat16),
            pltpu.VMEM((q, n), jnp.bfloat16),
            pltpu.VMEM((q, n), jnp.bfloat16),
            pltpu.VMEM((e, n), jnp.bfloat16),
            pltpu.VMEM((e, n), jnp.bfloat16),
            pltpu.VMEM((q, n), jnp.bfloat16),
            pltpu.VMEM((q, n), jnp.bfloat16),
            pltpu.SemaphoreType.DMA((8, G)),
            pltpu.SemaphoreType.DMA((8, G)),
        ],
        compiler_params=pltpu.CompilerParams(collective_id=0),
    )(x)
